# Initial kernel scaffold; baseline (speedup 1.0000x reference)
import jax
import jax.numpy as jnp
from jax.experimental import pallas as pl


def kernel(user_ids, W0, W1, W2, W3):
    raise NotImplementedError("write your pallas kernel here")


# speedup vs baseline: 1.9356x; 1.9356x over previous
import functools

import jax
import jax.numpy as jnp
from jax import lax
from jax.experimental import pallas as pl
from jax.experimental.pallas import tpu as pltpu
from jax.experimental.pallas import tpu_sc as plsc

_PSIZES = (41, 37, 31, 23)
_D = 16
_NT = 4
_B = 16384
_NC = 2
_NS = 16
_NW = _NC * _NS
_BPW = _B // _NW
_CHUNK = 128
_NCHUNK = _BPW // _CHUNK
_L = 16


def _body(uid_hbm, w0, w1, w2, w3, out_hbm, uid_v, idx_v, rows_v, sem):
    tables = (w0, w1, w2, w3)
    wid = lax.axis_index("s") * _NC + lax.axis_index("c")
    base = wid * _BPW

    pltpu.sync_copy(uid_hbm.at[pl.ds(base, _BPW)], uid_v)

    def mod_step(g, carry):
        u = uid_v[pl.ds(g * _L, _L)]
        for t, p in enumerate(_PSIZES):
            idx_v[t, pl.ds(g * _L, _L)] = u % p
        return carry

    lax.fori_loop(0, _BPW // _L, mod_step, 0)

    def fire(t):
        return [
            pltpu.async_copy(
                tables[t].at[idx_v.at[t, pl.ds(c * _CHUNK, _CHUNK)]],
                rows_v.at[t, pl.ds(c * _CHUNK, _CHUNK)],
                sem,
            )
            for c in range(_NCHUNK)
        ]

    pending = fire(0)
    for t in range(_NT):
        nxt = fire(t + 1) if t + 1 < _NT else []
        for cp in pending:
            cp.wait()
        pltpu.sync_copy(rows_v.at[t], out_hbm.at[pl.ds(base, _BPW), t])
        pending = nxt


@functools.partial(
    pl.kernel,
    out_type=jax.ShapeDtypeStruct((_B, _NT, _D), jnp.float32),
    mesh=plsc.VectorSubcoreMesh(core_axis_name="c", subcore_axis_name="s"),
    scratch_types=[
        pltpu.VMEM((_BPW,), jnp.int32),
        pltpu.VMEM((_NT, _BPW), jnp.int32),
        pltpu.VMEM((_NT, _BPW, _D), jnp.float32),
        pltpu.SemaphoreType.DMA,
    ],
    compiler_params=pltpu.CompilerParams(use_tc_tiling_on_sc=False),
)
def _sc_lookup(uid_hbm, w0, w1, w2, w3, out_hbm, uid_v, idx_v, rows_v, sem):
    _body(uid_hbm, w0, w1, w2, w3, out_hbm, uid_v, idx_v, rows_v, sem)


def kernel(user_ids, W0, W1, W2, W3):
    out = _sc_lookup(user_ids.astype(jnp.int32), W0, W1, W2, W3)
    return out.reshape(_B, _NT * _D)


# speedup vs baseline: 1.9944x; 1.0304x over previous
import functools

import jax
import jax.numpy as jnp
from jax import lax
from jax.experimental import pallas as pl
from jax.experimental.pallas import tpu as pltpu
from jax.experimental.pallas import tpu_sc as plsc

_PSIZES = (41, 37, 31, 23)
_D = 16
_NT = 4
_B = 16384
_NC = 2
_NS = 16
_NW = _NC * _NS
_BPW = _B // _NW
_CHUNK = 128
_NCHUNK = _BPW // _CHUNK
_L = 16


def _body(uid_hbm, w0, w1, w2, w3, out_hbm, uid_v, idx_v, rows_v, sem):
    tables = (w0, w1, w2, w3)
    wid = lax.axis_index("s") * _NC + lax.axis_index("c")
    base = wid * _BPW

    pltpu.sync_copy(uid_hbm.at[pl.ds(base, _BPW)], uid_v)

    def mod_step(g, carry):
        u = uid_v[pl.ds(g * _L, _L)]
        uf = u.astype(jnp.float32)
        for t, p in enumerate(_PSIZES):
            q = (uf * (1.0 / p)).astype(jnp.int32).astype(jnp.float32)
            r = uf - q * float(p)
            r = jnp.where(r < 0.0, r + p, r)
            r = jnp.where(r >= p, r - p, r)
            idx_v[t, pl.ds(g * _L, _L)] = r.astype(jnp.int32)
        return carry

    lax.fori_loop(0, _BPW // _L, mod_step, 0)

    def fire(t):
        return [
            pltpu.async_copy(
                tables[t].at[idx_v.at[t, pl.ds(c * _CHUNK, _CHUNK)]],
                rows_v.at[t, pl.ds(c * _CHUNK, _CHUNK)],
                sem,
            )
            for c in range(_NCHUNK)
        ]

    pending = fire(0)
    for t in range(_NT):
        nxt = fire(t + 1) if t + 1 < _NT else []
        for cp in pending:
            cp.wait()
        pltpu.sync_copy(rows_v.at[t], out_hbm.at[pl.ds(base, _BPW), t])
        pending = nxt


@functools.partial(
    pl.kernel,
    out_type=jax.ShapeDtypeStruct((_B, _NT, _D), jnp.float32),
    mesh=plsc.VectorSubcoreMesh(core_axis_name="c", subcore_axis_name="s"),
    scratch_types=[
        pltpu.VMEM((_BPW,), jnp.int32),
        pltpu.VMEM((_NT, _BPW), jnp.int32),
        pltpu.VMEM((_NT, _BPW, _D), jnp.float32),
        pltpu.SemaphoreType.DMA,
    ],
    compiler_params=pltpu.CompilerParams(use_tc_tiling_on_sc=False),
)
def _sc_lookup(uid_hbm, w0, w1, w2, w3, out_hbm, uid_v, idx_v, rows_v, sem):
    _body(uid_hbm, w0, w1, w2, w3, out_hbm, uid_v, idx_v, rows_v, sem)


def kernel(user_ids, W0, W1, W2, W3):
    out = _sc_lookup(user_ids.astype(jnp.int32), W0, W1, W2, W3)
    return out.reshape(_B, _NT * _D)
